# baseline (speedup 1.0000x reference)
import functools

import jax
import jax.numpy as jnp
from jax import lax
from jax.experimental import pallas as pl
from jax.experimental.pallas import tpu as pltpu
from jax.experimental.pallas import tpu_sc as plsc

BATCH = 16384
EMBED_DIM = 64
LANES = 16


def _sc_geometry():
    try:
        info = plsc.get_sparse_core_info()
        return info.num_cores, info.num_subcores
    except Exception:
        return 2, 16


def _body(rows_per_w, nc,
          uid_hbm, iid_hbm, uemb_hbm, iemb_hbm, ubias_hbm, ibias_hbm,
          out_hbm,
          uid_v, iid_v, urows_v, irows_v, ub_v, ib_v, out_v, t_v,
          sem):
    wid = lax.axis_index("s") * nc + lax.axis_index("c")
    base = wid * rows_per_w

    pltpu.sync_copy(uid_hbm.at[pl.ds(base, rows_per_w)], uid_v)
    pltpu.sync_copy(iid_hbm.at[pl.ds(base, rows_per_w)], iid_v)

    cp_u = pltpu.async_copy(uemb_hbm.at[uid_v], urows_v, sem)
    cp_i = pltpu.async_copy(iemb_hbm.at[iid_v], irows_v, sem)
    cp_ub = pltpu.async_copy(ubias_hbm.at[uid_v], ub_v, sem)
    cp_ib = pltpu.async_copy(ibias_hbm.at[iid_v], ib_v, sem)
    cp_u.wait()
    cp_i.wait()
    cp_ub.wait()
    cp_ib.wait()

    lane_iota = lax.iota(jnp.int32, LANES)

    def group(g, carry):
        base_r = g * LANES
        for r in range(LANES):
            acc = (urows_v[base_r + r, pl.ds(0, LANES)]
                   * irows_v[base_r + r, pl.ds(0, LANES)])
            for c in range(1, EMBED_DIM // LANES):
                acc = acc + (urows_v[base_r + r, pl.ds(c * LANES, LANES)]
                             * irows_v[base_r + r, pl.ds(c * LANES, LANES)])
            t_v[pl.ds(r * (LANES + 1), LANES)] = acc
        row_base = lane_iota * (LANES + 1)
        out16 = plsc.load_gather(t_v, [row_base])
        for c in range(1, LANES):
            out16 = out16 + plsc.load_gather(t_v, [row_base + c])
        out16 = out16 + ub_v[pl.ds(base_r, LANES)] + ib_v[pl.ds(base_r, LANES)]
        out_v[pl.ds(base_r, LANES)] = out16
        return carry

    lax.fori_loop(0, rows_per_w // LANES, group, 0)

    pltpu.sync_copy(out_v, out_hbm.at[pl.ds(base, rows_per_w)])


def kernel(user_ids, item_ids, user_embed, item_embed, user_bias, item_bias):
    nc, ns = _sc_geometry()
    nw = nc * ns
    rows_per_w = BATCH // nw

    mesh = plsc.VectorSubcoreMesh(core_axis_name="c", subcore_axis_name="s",
                                  num_cores=nc)

    run = pl.kernel(
        functools.partial(_body, rows_per_w, nc),
        out_type=jax.ShapeDtypeStruct((BATCH,), jnp.float32),
        mesh=mesh,
        scratch_types=[
            pltpu.VMEM((rows_per_w,), jnp.int32),
            pltpu.VMEM((rows_per_w,), jnp.int32),
            pltpu.VMEM((rows_per_w, EMBED_DIM), jnp.float32),
            pltpu.VMEM((rows_per_w, EMBED_DIM), jnp.float32),
            pltpu.VMEM((rows_per_w,), jnp.float32),
            pltpu.VMEM((rows_per_w,), jnp.float32),
            pltpu.VMEM((rows_per_w,), jnp.float32),
            pltpu.VMEM((LANES * (LANES + 1),), jnp.float32),
            pltpu.SemaphoreType.DMA,
        ],
        compiler_params=pltpu.CompilerParams(needs_layout_passes=False,
                                             use_tc_tiling_on_sc=False),
    )
    return run(user_ids, item_ids, user_embed, item_embed,
               user_bias.reshape(-1), item_bias.reshape(-1))


# speedup vs baseline: 1.4606x; 1.4606x over previous
import functools

import jax
import jax.numpy as jnp
from jax import lax
from jax.experimental import pallas as pl
from jax.experimental.pallas import tpu as pltpu
from jax.experimental.pallas import tpu_sc as plsc

NUM_ROWS = 1000000
BATCH = 16384
EMBED_DIM = 64
FUSED = 2 * EMBED_DIM
LANES = 16
CHUNKS = 2
TC_BLK = 2048


def _sc_geometry():
    try:
        info = plsc.get_sparse_core_info()
        return info.num_cores, info.num_subcores
    except Exception:
        return 2, 16


def _fuse_body(u_ref, i_ref, o_ref):
    o_ref[:, :EMBED_DIM] = u_ref[...].T
    o_ref[:, EMBED_DIM:] = i_ref[...].T


def _fuse_tables(user_embed_t, item_embed_t):
    return pl.pallas_call(
        _fuse_body,
        grid=(pl.cdiv(NUM_ROWS, TC_BLK),),
        in_specs=[
            pl.BlockSpec((EMBED_DIM, TC_BLK), lambda j: (0, j)),
            pl.BlockSpec((EMBED_DIM, TC_BLK), lambda j: (0, j)),
        ],
        out_specs=pl.BlockSpec((TC_BLK, FUSED), lambda j: (j, 0)),
        out_shape=jax.ShapeDtypeStruct((NUM_ROWS, FUSED), jnp.float32),
    )(user_embed_t, item_embed_t)


def _body(rows_per_w, nc,
          uid_hbm, iid_hbm, emb_hbm, ubias_hbm, ibias_hbm,
          out_hbm,
          uid_v, iid_v, urows_v, irows_v, ub_v, ib_v, out_v, t_v,
          sem):
    wid = lax.axis_index("s") * nc + lax.axis_index("c")
    base = wid * rows_per_w
    chunk = rows_per_w // CHUNKS

    pltpu.sync_copy(uid_hbm.at[pl.ds(base, rows_per_w)], uid_v)
    pltpu.sync_copy(iid_hbm.at[pl.ds(base, rows_per_w)], iid_v)

    cp_ub = pltpu.async_copy(ubias_hbm.at[uid_v], ub_v, sem)
    cp_ib = pltpu.async_copy(ibias_hbm.at[iid_v], ib_v, sem)

    lane_iota = lax.iota(jnp.int32, LANES)

    for ci in range(CHUNKS):
        cbase = ci * chunk
        cp_u = pltpu.async_copy(
            emb_hbm.at[uid_v.at[pl.ds(cbase, chunk)]], urows_v, sem)
        cp_i = pltpu.async_copy(
            emb_hbm.at[iid_v.at[pl.ds(cbase, chunk)]], irows_v, sem)
        cp_u.wait()
        cp_i.wait()

        def group(g, carry):
            base_r = g * LANES
            for r in range(LANES):
                acc = (urows_v[base_r + r, pl.ds(0, LANES)]
                       * irows_v[base_r + r, pl.ds(EMBED_DIM, LANES)])
                for c in range(1, EMBED_DIM // LANES):
                    acc = acc + (
                        urows_v[base_r + r, pl.ds(c * LANES, LANES)]
                        * irows_v[base_r + r,
                                  pl.ds(EMBED_DIM + c * LANES, LANES)])
                t_v[pl.ds(r * (LANES + 1), LANES)] = acc
            row_base = lane_iota * (LANES + 1)
            out16 = plsc.load_gather(t_v, [row_base])
            for c in range(1, LANES):
                out16 = out16 + plsc.load_gather(t_v, [row_base + c])
            out_v[pl.ds(cbase + base_r, LANES)] = out16
            return carry

        lax.fori_loop(0, chunk // LANES, group, 0)

    cp_ub.wait()
    cp_ib.wait()

    def bias_group(g, carry):
        base_r = g * LANES
        out16 = (out_v[pl.ds(base_r, LANES)]
                 + ub_v[pl.ds(base_r, LANES)]
                 + ib_v[pl.ds(base_r, LANES)])
        out_v[pl.ds(base_r, LANES)] = out16
        return carry

    lax.fori_loop(0, rows_per_w // LANES, bias_group, 0)

    pltpu.sync_copy(out_v, out_hbm.at[pl.ds(base, rows_per_w)])


def kernel(user_ids, item_ids, user_embed, item_embed, user_bias, item_bias):
    nc, ns = _sc_geometry()
    nw = nc * ns
    rows_per_w = BATCH // nw

    fused = _fuse_tables(user_embed.T, item_embed.T)

    mesh = plsc.VectorSubcoreMesh(core_axis_name="c", subcore_axis_name="s",
                                  num_cores=nc)

    run = pl.kernel(
        functools.partial(_body, rows_per_w, nc),
        out_type=jax.ShapeDtypeStruct((BATCH,), jnp.float32),
        mesh=mesh,
        scratch_types=[
            pltpu.VMEM((rows_per_w,), jnp.int32),
            pltpu.VMEM((rows_per_w,), jnp.int32),
            pltpu.VMEM((rows_per_w // CHUNKS, FUSED), jnp.float32),
            pltpu.VMEM((rows_per_w // CHUNKS, FUSED), jnp.float32),
            pltpu.VMEM((rows_per_w,), jnp.float32),
            pltpu.VMEM((rows_per_w,), jnp.float32),
            pltpu.VMEM((rows_per_w,), jnp.float32),
            pltpu.VMEM((LANES * (LANES + 1),), jnp.float32),
            pltpu.SemaphoreType.DMA,
        ],
        compiler_params=pltpu.CompilerParams(needs_layout_passes=False,
                                             use_tc_tiling_on_sc=False),
    )
    return run(user_ids, item_ids, fused,
               user_bias.reshape(-1), item_bias.reshape(-1))


# speedup vs baseline: 1.7244x; 1.1807x over previous
import functools

import jax
import jax.numpy as jnp
from jax import lax
from jax.experimental import pallas as pl
from jax.experimental.pallas import tpu as pltpu
from jax.experimental.pallas import tpu_sc as plsc

NUM_ROWS = 1000000
BATCH = 16384
EMBED_DIM = 64
FUSED = 2 * EMBED_DIM
LANES = 16
CHUNKS = 2
TC_BLK = 2048


def _sc_geometry():
    try:
        info = plsc.get_sparse_core_info()
        return info.num_cores, info.num_subcores
    except Exception:
        return 2, 16


def _fuse_body(u_ref, i_ref, o_ref):
    o_ref[...] = jnp.concatenate([u_ref[...], i_ref[...]], axis=0).T


def _fuse_tables(user_embed_t, item_embed_t):
    return pl.pallas_call(
        _fuse_body,
        grid=(pl.cdiv(NUM_ROWS, TC_BLK),),
        in_specs=[
            pl.BlockSpec((EMBED_DIM, TC_BLK), lambda j: (0, j)),
            pl.BlockSpec((EMBED_DIM, TC_BLK), lambda j: (0, j)),
        ],
        out_specs=pl.BlockSpec((TC_BLK, FUSED), lambda j: (j, 0)),
        out_shape=jax.ShapeDtypeStruct((NUM_ROWS, FUSED), jnp.float32),
    )(user_embed_t, item_embed_t)


def _body(rows_per_w, nc,
          uid_hbm, iid_hbm, emb_hbm, ubias_hbm, ibias_hbm,
          out_hbm,
          uid_v, iid_v, urows_v, irows_v, ub_v, ib_v, out_v, t_v,
          sem):
    wid = lax.axis_index("s") * nc + lax.axis_index("c")
    base = wid * rows_per_w
    chunk = rows_per_w // CHUNKS

    pltpu.sync_copy(uid_hbm.at[pl.ds(base, rows_per_w)], uid_v)
    pltpu.sync_copy(iid_hbm.at[pl.ds(base, rows_per_w)], iid_v)

    cp_ub = pltpu.async_copy(ubias_hbm.at[uid_v], ub_v, sem)
    cp_ib = pltpu.async_copy(ibias_hbm.at[iid_v], ib_v, sem)

    lane_iota = lax.iota(jnp.int32, LANES)

    for ci in range(CHUNKS):
        cbase = ci * chunk
        cp_u = pltpu.async_copy(
            emb_hbm.at[uid_v.at[pl.ds(cbase, chunk)]], urows_v, sem)
        cp_i = pltpu.async_copy(
            emb_hbm.at[iid_v.at[pl.ds(cbase, chunk)]], irows_v, sem)
        cp_u.wait()
        cp_i.wait()

        def group(g, carry):
            base_r = g * LANES
            for r in range(LANES):
                acc = (urows_v[base_r + r, pl.ds(0, LANES)]
                       * irows_v[base_r + r, pl.ds(EMBED_DIM, LANES)])
                for c in range(1, EMBED_DIM // LANES):
                    acc = acc + (
                        urows_v[base_r + r, pl.ds(c * LANES, LANES)]
                        * irows_v[base_r + r,
                                  pl.ds(EMBED_DIM + c * LANES, LANES)])
                t_v[pl.ds(r * (LANES + 1), LANES)] = acc
            row_base = lane_iota * (LANES + 1)
            out16 = plsc.load_gather(t_v, [row_base])
            for c in range(1, LANES):
                out16 = out16 + plsc.load_gather(t_v, [row_base + c])
            out_v[pl.ds(cbase + base_r, LANES)] = out16
            return carry

        lax.fori_loop(0, chunk // LANES, group, 0)

    cp_ub.wait()
    cp_ib.wait()

    def bias_group(g, carry):
        base_r = g * LANES
        out16 = (out_v[pl.ds(base_r, LANES)]
                 + ub_v[pl.ds(base_r, LANES)]
                 + ib_v[pl.ds(base_r, LANES)])
        out_v[pl.ds(base_r, LANES)] = out16
        return carry

    lax.fori_loop(0, rows_per_w // LANES, bias_group, 0)

    pltpu.sync_copy(out_v, out_hbm.at[pl.ds(base, rows_per_w)])


def kernel(user_ids, item_ids, user_embed, item_embed, user_bias, item_bias):
    nc, ns = _sc_geometry()
    nw = nc * ns
    rows_per_w = BATCH // nw

    fused = _fuse_tables(user_embed.T, item_embed.T)

    mesh = plsc.VectorSubcoreMesh(core_axis_name="c", subcore_axis_name="s",
                                  num_cores=nc)

    run = pl.kernel(
        functools.partial(_body, rows_per_w, nc),
        out_type=jax.ShapeDtypeStruct((BATCH,), jnp.float32),
        mesh=mesh,
        scratch_types=[
            pltpu.VMEM((rows_per_w,), jnp.int32),
            pltpu.VMEM((rows_per_w,), jnp.int32),
            pltpu.VMEM((rows_per_w // CHUNKS, FUSED), jnp.float32),
            pltpu.VMEM((rows_per_w // CHUNKS, FUSED), jnp.float32),
            pltpu.VMEM((rows_per_w,), jnp.float32),
            pltpu.VMEM((rows_per_w,), jnp.float32),
            pltpu.VMEM((rows_per_w,), jnp.float32),
            pltpu.VMEM((LANES * (LANES + 1),), jnp.float32),
            pltpu.SemaphoreType.DMA,
        ],
        compiler_params=pltpu.CompilerParams(needs_layout_passes=False,
                                             use_tc_tiling_on_sc=False),
    )
    return run(user_ids, item_ids, fused,
               user_bias.reshape(-1), item_bias.reshape(-1))


# speedup vs baseline: 2.4950x; 1.4469x over previous
import functools

import jax
import jax.numpy as jnp
from jax import lax
from jax.experimental import pallas as pl
from jax.experimental.pallas import tpu as pltpu
from jax.experimental.pallas import tpu_sc as plsc

NUM_ROWS = 1000000
BATCH = 16384
EMBED_DIM = 64
FUSED = 2 * EMBED_DIM
LANES = 16
CHUNKS = 2
TC_BLK = 8192


def _sc_geometry():
    try:
        info = plsc.get_sparse_core_info()
        return info.num_cores, info.num_subcores
    except Exception:
        return 2, 16


def _fuse_body(u_ref, i_ref, o_ref):
    o_ref[...] = jnp.concatenate([u_ref[...], i_ref[...]], axis=0).T


def _fuse_tables(user_embed_t, item_embed_t):
    return pl.pallas_call(
        _fuse_body,
        grid=(pl.cdiv(NUM_ROWS, TC_BLK),),
        in_specs=[
            pl.BlockSpec((EMBED_DIM, TC_BLK), lambda j: (0, j)),
            pl.BlockSpec((EMBED_DIM, TC_BLK), lambda j: (0, j)),
        ],
        out_specs=pl.BlockSpec((TC_BLK, FUSED), lambda j: (j, 0)),
        out_shape=jax.ShapeDtypeStruct((NUM_ROWS, FUSED), jnp.float32),
    )(user_embed_t, item_embed_t)


def _body(rows_per_w, nc,
          uid_hbm, iid_hbm, emb_hbm, ubias_hbm, ibias_hbm,
          out_hbm,
          uid_v, iid_v, urows_v, irows_v, ub_v, ib_v, out_v, t_v,
          sem):
    wid = lax.axis_index("s") * nc + lax.axis_index("c")
    base = wid * rows_per_w
    chunk = rows_per_w // CHUNKS

    pltpu.sync_copy(uid_hbm.at[pl.ds(base, rows_per_w)], uid_v)
    pltpu.sync_copy(iid_hbm.at[pl.ds(base, rows_per_w)], iid_v)

    cp_ub = pltpu.async_copy(ubias_hbm.at[uid_v], ub_v, sem)
    cp_ib = pltpu.async_copy(ibias_hbm.at[iid_v], ib_v, sem)

    lane_iota = lax.iota(jnp.int32, LANES)

    for ci in range(CHUNKS):
        cbase = ci * chunk
        cp_u = pltpu.async_copy(
            emb_hbm.at[uid_v.at[pl.ds(cbase, chunk)]], urows_v, sem)
        cp_i = pltpu.async_copy(
            emb_hbm.at[iid_v.at[pl.ds(cbase, chunk)]], irows_v, sem)
        cp_u.wait()
        cp_i.wait()

        def group(g, carry):
            base_r = g * LANES
            for r in range(LANES):
                acc = (urows_v[base_r + r, pl.ds(0, LANES)]
                       * irows_v[base_r + r, pl.ds(EMBED_DIM, LANES)])
                for c in range(1, EMBED_DIM // LANES):
                    acc = acc + (
                        urows_v[base_r + r, pl.ds(c * LANES, LANES)]
                        * irows_v[base_r + r,
                                  pl.ds(EMBED_DIM + c * LANES, LANES)])
                t_v[pl.ds(r * (LANES + 1), LANES)] = acc
            row_base = lane_iota * (LANES + 1)
            out16 = plsc.load_gather(t_v, [row_base])
            for c in range(1, LANES):
                out16 = out16 + plsc.load_gather(t_v, [row_base + c])
            out_v[pl.ds(cbase + base_r, LANES)] = out16
            return carry

        lax.fori_loop(0, chunk // LANES, group, 0)

    cp_ub.wait()
    cp_ib.wait()

    def bias_group(g, carry):
        base_r = g * LANES
        out16 = (out_v[pl.ds(base_r, LANES)]
                 + ub_v[pl.ds(base_r, LANES)]
                 + ib_v[pl.ds(base_r, LANES)])
        out_v[pl.ds(base_r, LANES)] = out16
        return carry

    lax.fori_loop(0, rows_per_w // LANES, bias_group, 0)

    pltpu.sync_copy(out_v, out_hbm.at[pl.ds(base, rows_per_w)])


def kernel(user_ids, item_ids, user_embed, item_embed, user_bias, item_bias):
    nc, ns = _sc_geometry()
    nw = nc * ns
    rows_per_w = BATCH // nw

    fused = _fuse_tables(user_embed.T, item_embed.T)

    mesh = plsc.VectorSubcoreMesh(core_axis_name="c", subcore_axis_name="s",
                                  num_cores=nc)

    run = pl.kernel(
        functools.partial(_body, rows_per_w, nc),
        out_type=jax.ShapeDtypeStruct((BATCH,), jnp.float32),
        mesh=mesh,
        scratch_types=[
            pltpu.VMEM((rows_per_w,), jnp.int32),
            pltpu.VMEM((rows_per_w,), jnp.int32),
            pltpu.VMEM((rows_per_w // CHUNKS, FUSED), jnp.float32),
            pltpu.VMEM((rows_per_w // CHUNKS, FUSED), jnp.float32),
            pltpu.VMEM((rows_per_w,), jnp.float32),
            pltpu.VMEM((rows_per_w,), jnp.float32),
            pltpu.VMEM((rows_per_w,), jnp.float32),
            pltpu.VMEM((LANES * (LANES + 1),), jnp.float32),
            pltpu.SemaphoreType.DMA,
        ],
        compiler_params=pltpu.CompilerParams(needs_layout_passes=False,
                                             use_tc_tiling_on_sc=False),
    )
    return run(user_ids, item_ids, fused,
               user_bias.reshape(-1), item_bias.reshape(-1))


# speedup vs baseline: 2.5393x; 1.0178x over previous
import functools

import jax
import jax.numpy as jnp
from jax import lax
from jax.experimental import pallas as pl
from jax.experimental.pallas import tpu as pltpu
from jax.experimental.pallas import tpu_sc as plsc

NUM_ROWS = 1000000
BATCH = 16384
EMBED_DIM = 64
FUSED = 2 * EMBED_DIM
LANES = 16
CHUNKS = 2
TC_BLK = 16384


def _sc_geometry():
    try:
        info = plsc.get_sparse_core_info()
        return info.num_cores, info.num_subcores
    except Exception:
        return 2, 16


def _fuse_body(u_ref, i_ref, o_ref):
    o_ref[...] = jnp.concatenate([u_ref[...], i_ref[...]], axis=0).T


def _fuse_tables(user_embed_t, item_embed_t):
    return pl.pallas_call(
        _fuse_body,
        grid=(pl.cdiv(NUM_ROWS, TC_BLK),),
        in_specs=[
            pl.BlockSpec((EMBED_DIM, TC_BLK), lambda j: (0, j)),
            pl.BlockSpec((EMBED_DIM, TC_BLK), lambda j: (0, j)),
        ],
        out_specs=pl.BlockSpec((TC_BLK, FUSED), lambda j: (j, 0)),
        out_shape=jax.ShapeDtypeStruct((NUM_ROWS, FUSED), jnp.float32),
    )(user_embed_t, item_embed_t)


def _body(rows_per_w, nc,
          uid_hbm, iid_hbm, emb_hbm, ubias_hbm, ibias_hbm,
          out_hbm,
          uid_v, iid_v, urows_v, irows_v, ub_v, ib_v, out_v, t_v,
          sem):
    wid = lax.axis_index("s") * nc + lax.axis_index("c")
    base = wid * rows_per_w
    chunk = rows_per_w // CHUNKS

    pltpu.sync_copy(uid_hbm.at[pl.ds(base, rows_per_w)], uid_v)
    pltpu.sync_copy(iid_hbm.at[pl.ds(base, rows_per_w)], iid_v)

    cp_ub = pltpu.async_copy(ubias_hbm.at[uid_v], ub_v, sem)
    cp_ib = pltpu.async_copy(ibias_hbm.at[iid_v], ib_v, sem)

    lane_iota = lax.iota(jnp.int32, LANES)

    for ci in range(CHUNKS):
        cbase = ci * chunk
        cp_u = pltpu.async_copy(
            emb_hbm.at[uid_v.at[pl.ds(cbase, chunk)]], urows_v, sem)
        cp_i = pltpu.async_copy(
            emb_hbm.at[iid_v.at[pl.ds(cbase, chunk)]], irows_v, sem)
        cp_u.wait()
        cp_i.wait()

        def group(g, carry):
            base_r = g * LANES
            for r in range(LANES):
                acc = (urows_v[base_r + r, pl.ds(0, LANES)]
                       * irows_v[base_r + r, pl.ds(EMBED_DIM, LANES)])
                for c in range(1, EMBED_DIM // LANES):
                    acc = acc + (
                        urows_v[base_r + r, pl.ds(c * LANES, LANES)]
                        * irows_v[base_r + r,
                                  pl.ds(EMBED_DIM + c * LANES, LANES)])
                t_v[pl.ds(r * (LANES + 1), LANES)] = acc
            row_base = lane_iota * (LANES + 1)
            out16 = plsc.load_gather(t_v, [row_base])
            for c in range(1, LANES):
                out16 = out16 + plsc.load_gather(t_v, [row_base + c])
            out_v[pl.ds(cbase + base_r, LANES)] = out16
            return carry

        lax.fori_loop(0, chunk // LANES, group, 0)

    cp_ub.wait()
    cp_ib.wait()

    def bias_group(g, carry):
        base_r = g * LANES
        out16 = (out_v[pl.ds(base_r, LANES)]
                 + ub_v[pl.ds(base_r, LANES)]
                 + ib_v[pl.ds(base_r, LANES)])
        out_v[pl.ds(base_r, LANES)] = out16
        return carry

    lax.fori_loop(0, rows_per_w // LANES, bias_group, 0)

    pltpu.sync_copy(out_v, out_hbm.at[pl.ds(base, rows_per_w)])


def kernel(user_ids, item_ids, user_embed, item_embed, user_bias, item_bias):
    nc, ns = _sc_geometry()
    nw = nc * ns
    rows_per_w = BATCH // nw

    fused = _fuse_tables(user_embed.T, item_embed.T)

    mesh = plsc.VectorSubcoreMesh(core_axis_name="c", subcore_axis_name="s",
                                  num_cores=nc)

    run = pl.kernel(
        functools.partial(_body, rows_per_w, nc),
        out_type=jax.ShapeDtypeStruct((BATCH,), jnp.float32),
        mesh=mesh,
        scratch_types=[
            pltpu.VMEM((rows_per_w,), jnp.int32),
            pltpu.VMEM((rows_per_w,), jnp.int32),
            pltpu.VMEM((rows_per_w // CHUNKS, FUSED), jnp.float32),
            pltpu.VMEM((rows_per_w // CHUNKS, FUSED), jnp.float32),
            pltpu.VMEM((rows_per_w,), jnp.float32),
            pltpu.VMEM((rows_per_w,), jnp.float32),
            pltpu.VMEM((rows_per_w,), jnp.float32),
            pltpu.VMEM((LANES * (LANES + 1),), jnp.float32),
            pltpu.SemaphoreType.DMA,
        ],
        compiler_params=pltpu.CompilerParams(needs_layout_passes=False,
                                             use_tc_tiling_on_sc=False),
    )
    return run(user_ids, item_ids, fused,
               user_bias.reshape(-1), item_bias.reshape(-1))


# speedup vs baseline: 2.5441x; 1.0019x over previous
import functools

import jax
import jax.numpy as jnp
from jax import lax
from jax.experimental import pallas as pl
from jax.experimental.pallas import tpu as pltpu
from jax.experimental.pallas import tpu_sc as plsc

NUM_ROWS = 1000000
BATCH = 16384
EMBED_DIM = 64
FUSED = 2 * EMBED_DIM
LANES = 16
CHUNKS = 2
TC_BLK = 25600


def _sc_geometry():
    try:
        info = plsc.get_sparse_core_info()
        return info.num_cores, info.num_subcores
    except Exception:
        return 2, 16


def _fuse_body(u_ref, i_ref, o_ref):
    o_ref[...] = jnp.concatenate([u_ref[...], i_ref[...]], axis=0).T


def _fuse_tables(user_embed_t, item_embed_t):
    return pl.pallas_call(
        _fuse_body,
        grid=(pl.cdiv(NUM_ROWS, TC_BLK),),
        in_specs=[
            pl.BlockSpec((EMBED_DIM, TC_BLK), lambda j: (0, j)),
            pl.BlockSpec((EMBED_DIM, TC_BLK), lambda j: (0, j)),
        ],
        out_specs=pl.BlockSpec((TC_BLK, FUSED), lambda j: (j, 0)),
        out_shape=jax.ShapeDtypeStruct((NUM_ROWS, FUSED), jnp.float32),
    )(user_embed_t, item_embed_t)


def _body(rows_per_w, nc,
          uid_hbm, iid_hbm, emb_hbm, ubias_hbm, ibias_hbm,
          out_hbm,
          uid_v, iid_v, urows_v, irows_v, ub_v, ib_v, out_v, t_v,
          sem):
    wid = lax.axis_index("s") * nc + lax.axis_index("c")
    base = wid * rows_per_w
    chunk = rows_per_w // CHUNKS

    pltpu.sync_copy(uid_hbm.at[pl.ds(base, rows_per_w)], uid_v)
    pltpu.sync_copy(iid_hbm.at[pl.ds(base, rows_per_w)], iid_v)

    cp_ub = pltpu.async_copy(ubias_hbm.at[uid_v], ub_v, sem)
    cp_ib = pltpu.async_copy(ibias_hbm.at[iid_v], ib_v, sem)

    lane_iota = lax.iota(jnp.int32, LANES)

    for ci in range(CHUNKS):
        cbase = ci * chunk
        cp_u = pltpu.async_copy(
            emb_hbm.at[uid_v.at[pl.ds(cbase, chunk)]], urows_v, sem)
        cp_i = pltpu.async_copy(
            emb_hbm.at[iid_v.at[pl.ds(cbase, chunk)]], irows_v, sem)
        cp_u.wait()
        cp_i.wait()

        def group(g, carry):
            base_r = g * LANES
            for r in range(LANES):
                acc = (urows_v[base_r + r, pl.ds(0, LANES)]
                       * irows_v[base_r + r, pl.ds(EMBED_DIM, LANES)])
                for c in range(1, EMBED_DIM // LANES):
                    acc = acc + (
                        urows_v[base_r + r, pl.ds(c * LANES, LANES)]
                        * irows_v[base_r + r,
                                  pl.ds(EMBED_DIM + c * LANES, LANES)])
                t_v[pl.ds(r * (LANES + 1), LANES)] = acc
            row_base = lane_iota * (LANES + 1)
            out16 = plsc.load_gather(t_v, [row_base])
            for c in range(1, LANES):
                out16 = out16 + plsc.load_gather(t_v, [row_base + c])
            out_v[pl.ds(cbase + base_r, LANES)] = out16
            return carry

        lax.fori_loop(0, chunk // LANES, group, 0)

    cp_ub.wait()
    cp_ib.wait()

    def bias_group(g, carry):
        base_r = g * LANES
        out16 = (out_v[pl.ds(base_r, LANES)]
                 + ub_v[pl.ds(base_r, LANES)]
                 + ib_v[pl.ds(base_r, LANES)])
        out_v[pl.ds(base_r, LANES)] = out16
        return carry

    lax.fori_loop(0, rows_per_w // LANES, bias_group, 0)

    pltpu.sync_copy(out_v, out_hbm.at[pl.ds(base, rows_per_w)])


def kernel(user_ids, item_ids, user_embed, item_embed, user_bias, item_bias):
    nc, ns = _sc_geometry()
    nw = nc * ns
    rows_per_w = BATCH // nw

    fused = _fuse_tables(user_embed.T, item_embed.T)

    mesh = plsc.VectorSubcoreMesh(core_axis_name="c", subcore_axis_name="s",
                                  num_cores=nc)

    run = pl.kernel(
        functools.partial(_body, rows_per_w, nc),
        out_type=jax.ShapeDtypeStruct((BATCH,), jnp.float32),
        mesh=mesh,
        scratch_types=[
            pltpu.VMEM((rows_per_w,), jnp.int32),
            pltpu.VMEM((rows_per_w,), jnp.int32),
            pltpu.VMEM((rows_per_w // CHUNKS, FUSED), jnp.float32),
            pltpu.VMEM((rows_per_w // CHUNKS, FUSED), jnp.float32),
            pltpu.VMEM((rows_per_w,), jnp.float32),
            pltpu.VMEM((rows_per_w,), jnp.float32),
            pltpu.VMEM((rows_per_w,), jnp.float32),
            pltpu.VMEM((LANES * (LANES + 1),), jnp.float32),
            pltpu.SemaphoreType.DMA,
        ],
        compiler_params=pltpu.CompilerParams(needs_layout_passes=False,
                                             use_tc_tiling_on_sc=False),
    )
    return run(user_ids, item_ids, fused,
               user_bias.reshape(-1), item_bias.reshape(-1))
